# baseline (speedup 1.0000x reference)
import jax
import jax.numpy as jnp
import numpy as np
from jax.experimental import pallas as pl
from jax.sharding import Mesh, PartitionSpec as P

HIDDEN = 2048
EXPERTS = 64
K = 8
ROW_BLK = 1024


def _gate_kernel(hs_ref, w_ref, b_ref, idx_ref, wgt_ref):
    logits = jax.lax.dot_general(
        hs_ref[...], w_ref[...],
        dimension_numbers=(((1,), (1,)), ((), ())),
        preferred_element_type=jnp.float32,
    ) + b_ref[...]

    vals = logits.T
    rows = vals.shape[1]
    eidx = jax.lax.broadcasted_iota(jnp.int32, (EXPERTS, rows), 0)
    top_vals = []
    top_idx = []
    for _ in range(K):
        m = jnp.max(vals, axis=0, keepdims=True)
        i = jnp.min(jnp.where(vals == m, eidx, EXPERTS), axis=0, keepdims=True)
        top_vals.append(m)
        top_idx.append(i)
        vals = jnp.where(eidx == i, -jnp.inf, vals)
    tv = jnp.concatenate(top_vals, axis=0)
    ti = jnp.concatenate(top_idx, axis=0)
    e = jnp.exp(tv - tv[0:1, :])
    w = e / jnp.sum(e, axis=0, keepdims=True)
    idx_ref[...] = ti
    wgt_ref[...] = w


def _gate_call(hs, w, b):
    n = hs.shape[0]
    grid = (n // ROW_BLK,)
    return pl.pallas_call(
        _gate_kernel,
        grid=grid,
        in_specs=[
            pl.BlockSpec((ROW_BLK, HIDDEN), lambda i: (i, 0)),
            pl.BlockSpec((EXPERTS, HIDDEN), lambda i: (0, 0)),
            pl.BlockSpec((1, EXPERTS), lambda i: (0, 0)),
        ],
        out_specs=[
            pl.BlockSpec((K, ROW_BLK), lambda i: (0, i)),
            pl.BlockSpec((K, ROW_BLK), lambda i: (0, i)),
        ],
        out_shape=[
            jax.ShapeDtypeStruct((K, n), jnp.int32),
            jax.ShapeDtypeStruct((K, n), jnp.float32),
        ],
    )(hs, w, b)


def kernel(hidden_states, weight, bias):
    batch, seq, hidden = hidden_states.shape
    n = batch * seq
    hs = hidden_states.reshape(n, hidden)
    b = bias.reshape(1, EXPERTS)

    devs = jax.devices()
    ndev = len(devs)
    while ndev > 1 and (n % (ndev * ROW_BLK)) != 0:
        ndev -= 1
    if ndev > 1:
        mesh = Mesh(np.array(devs[:ndev]), ("x",))
        idx, wgt = jax.shard_map(
            _gate_call,
            mesh=mesh,
            in_specs=(P("x", None), P(None, None), P(None, None)),
            out_specs=(P(None, "x"), P(None, "x")),
            check_vma=False,
        )(hs, weight, b)
    else:
        idx, wgt = _gate_call(hs, weight, b)
    return (idx.T, wgt.T)


# speedup vs baseline: 6.7252x; 6.7252x over previous
import dataclasses

import jax
import jax.numpy as jnp
from jax.experimental import pallas as pl
from jax.experimental.pallas import tpu as pltpu
from jax.experimental.pallas import tpu_sc as plsc

HIDDEN = 2048
EXPERTS = 64
K = 8
ROW_BLK = 1024
SC_ROWS = 16


def _logits_kernel(hs_ref, w_ref, b_ref, out_ref):
    out_ref[...] = jax.lax.dot_general(
        hs_ref[...], w_ref[...],
        dimension_numbers=(((1,), (1,)), ((), ())),
        preferred_element_type=jnp.float32,
    ) + b_ref[...]


def _logits_call(hs, w, b):
    n = hs.shape[0]
    return pl.pallas_call(
        _logits_kernel,
        grid=(n // ROW_BLK,),
        in_specs=[
            pl.BlockSpec((ROW_BLK, HIDDEN), lambda i: (i, 0)),
            pl.BlockSpec((EXPERTS, HIDDEN), lambda i: (0, 0)),
            pl.BlockSpec((1, EXPERTS), lambda i: (0, 0)),
        ],
        out_specs=pl.BlockSpec((ROW_BLK, EXPERTS), lambda i: (i, 0)),
        out_shape=jax.ShapeDtypeStruct((n, EXPERTS), jnp.float32),
    )(hs, w, b)


def _merge(ka, va, kb, vb):
    sel = ka >= kb
    return jnp.where(sel, ka, kb), jnp.where(sel, va, vb)


def _sc_topk(logits):
    n = logits.shape[0]
    mesh = plsc.VectorSubcoreMesh(core_axis_name="core",
                                  subcore_axis_name="subcore")

    cp = pltpu.CompilerParams()
    if "needs_layout_passes" in pltpu.CompilerParams.__dataclass_fields__:
        cp = dataclasses.replace(cp, needs_layout_passes=False)

    @pl.kernel(
        out_type=[
            jax.ShapeDtypeStruct((n, 16), jnp.int32),
            jax.ShapeDtypeStruct((n, 16), jnp.float32),
        ],
        mesh=mesh,
        compiler_params=cp,
    )
    def sc_kernel(logits_hbm, idx_hbm, wgt_hbm):
        def body(in_vmem, idx_vmem, wgt_vmem):
            lane = jax.lax.iota(jnp.int32, 16)

            @pl.loop(0, SC_ROWS)
            def _(r):
                c0 = in_vmem[r, pl.ds(0, 16)]
                c1 = in_vmem[r, pl.ds(16, 16)]
                c2 = in_vmem[r, pl.ds(32, 16)]
                c3 = in_vmem[r, pl.ds(48, 16)]
                k0, v0 = plsc.sort_key_val(c0, lane)
                k1, v1 = plsc.sort_key_val(c1, lane + 16, descending=True)
                k2, v2 = plsc.sort_key_val(c2, lane + 32)
                k3, v3 = plsc.sort_key_val(c3, lane + 48, descending=True)
                m01, i01 = _merge(k0, v0, k1, v1)
                m23, i23 = _merge(k2, v2, k3, v3)
                k01, v01 = plsc.sort_key_val(m01, i01)
                k23, v23 = plsc.sort_key_val(m23, i23, descending=True)
                mt, it = _merge(k01, v01, k23, v23)
                ks, vs = plsc.sort_key_val(mt, it, descending=True)
                e = jnp.exp(ks - jnp.max(ks))
                e = jnp.where(lane < K, e, 0.0)
                w = e / jnp.sum(e)
                idx_vmem[r, :] = vs
                wgt_vmem[r, :] = w

        pltpu.emit_pipeline(
            body,
            grid=(n // SC_ROWS,),
            in_specs=[pl.BlockSpec((SC_ROWS, EXPERTS), lambda i: (i, 0))],
            out_specs=[
                pl.BlockSpec((SC_ROWS, 16), lambda i: (i, 0)),
                pl.BlockSpec((SC_ROWS, 16), lambda i: (i, 0)),
            ],
            core_axis_name=("core", "subcore"),
            dimension_semantics=(pltpu.PARALLEL,),
        )(logits_hbm, idx_hbm, wgt_hbm)

    return sc_kernel(logits)


def kernel(hidden_states, weight, bias):
    batch, seq, hidden = hidden_states.shape
    n = batch * seq
    hs = hidden_states.reshape(n, hidden)
    b = bias.reshape(1, EXPERTS)
    logits = _logits_call(hs, weight, b)
    idxp, wgtp = _sc_topk(logits)
    return (idxp[:, :K], wgtp[:, :K])


# speedup vs baseline: 16.6909x; 2.4818x over previous
import jax
import jax.numpy as jnp
from jax.experimental import pallas as pl

HIDDEN = 2048
EXPERTS = 64
K = 8
ROW_BLK = 1024


def _gate_kernel(hs_ref, w_ref, b_ref, idx_ref, wgt_ref):
    logits = jax.lax.dot_general(
        hs_ref[...], w_ref[...],
        dimension_numbers=(((1,), (1,)), ((), ())),
        preferred_element_type=jnp.float32,
    ) + b_ref[...]

    vals = logits.T
    rows = vals.shape[1]
    eidx = jax.lax.broadcasted_iota(jnp.int32, (EXPERTS, rows), 0)
    top_vals = []
    top_idx = []
    for _ in range(K):
        m = jnp.max(vals, axis=0, keepdims=True)
        i = jnp.min(jnp.where(vals == m, eidx, EXPERTS), axis=0, keepdims=True)
        top_vals.append(m)
        top_idx.append(i)
        vals = jnp.where(eidx == i, -jnp.inf, vals)
    tv = jnp.concatenate(top_vals, axis=0)
    ti = jnp.concatenate(top_idx, axis=0)
    e = jnp.exp(tv - tv[0:1, :])
    w = e / jnp.sum(e, axis=0, keepdims=True)
    idx_ref[...] = ti
    wgt_ref[...] = w


def kernel(hidden_states, weight, bias):
    batch, seq, hidden = hidden_states.shape
    n = batch * seq
    hs = hidden_states.reshape(n, hidden)
    b = bias.reshape(1, EXPERTS)

    grid = (n // ROW_BLK,)
    idx, wgt = pl.pallas_call(
        _gate_kernel,
        grid=grid,
        in_specs=[
            pl.BlockSpec((ROW_BLK, HIDDEN), lambda i: (i, 0)),
            pl.BlockSpec((EXPERTS, HIDDEN), lambda i: (0, 0)),
            pl.BlockSpec((1, EXPERTS), lambda i: (0, 0)),
        ],
        out_specs=[
            pl.BlockSpec((K, ROW_BLK), lambda i: (0, i)),
            pl.BlockSpec((K, ROW_BLK), lambda i: (0, i)),
        ],
        out_shape=[
            jax.ShapeDtypeStruct((K, n), jnp.int32),
            jax.ShapeDtypeStruct((K, n), jnp.float32),
        ],
    )(hs, weight, b)
    return (idx.T, wgt.T)


# speedup vs baseline: 16.7039x; 1.0008x over previous
import jax
import jax.numpy as jnp
from jax.experimental import pallas as pl
from jax.experimental.pallas import tpu as pltpu

HIDDEN = 2048
EXPERTS = 64
K = 8
ROW_BLK = 1024


def _gate_kernel(hs_ref, w_ref, b_ref, idx_ref, wgt_ref):
    logits = jax.lax.dot_general(
        hs_ref[...], w_ref[...],
        dimension_numbers=(((1,), (1,)), ((), ())),
        preferred_element_type=jnp.float32,
    ) + b_ref[...]

    vals = logits.T
    rows = vals.shape[1]
    eidx = jax.lax.broadcasted_iota(jnp.int32, (EXPERTS, rows), 0)
    top_vals = []
    top_idx = []
    for _ in range(K):
        m = jnp.max(vals, axis=0, keepdims=True)
        i = jnp.min(jnp.where(vals == m, eidx, EXPERTS), axis=0, keepdims=True)
        top_vals.append(m)
        top_idx.append(i)
        vals = jnp.where(eidx == i, -jnp.inf, vals)
    tv = jnp.concatenate(top_vals, axis=0)
    ti = jnp.concatenate(top_idx, axis=0)
    e = jnp.exp(tv - tv[0:1, :])
    w = e / jnp.sum(e, axis=0, keepdims=True)
    idx_ref[...] = ti
    wgt_ref[...] = w


def kernel(hidden_states, weight, bias):
    batch, seq, hidden = hidden_states.shape
    n = batch * seq
    hs = hidden_states.reshape(n, hidden)
    b = bias.reshape(1, EXPERTS)

    grid = (n // ROW_BLK,)
    idx, wgt = pl.pallas_call(
        _gate_kernel,
        grid=grid,
        in_specs=[
            pl.BlockSpec((ROW_BLK, HIDDEN), lambda i: (i, 0)),
            pl.BlockSpec((EXPERTS, HIDDEN), lambda i: (0, 0)),
            pl.BlockSpec((1, EXPERTS), lambda i: (0, 0)),
        ],
        out_specs=[
            pl.BlockSpec((K, ROW_BLK), lambda i: (0, i)),
            pl.BlockSpec((K, ROW_BLK), lambda i: (0, i)),
        ],
        out_shape=[
            jax.ShapeDtypeStruct((K, n), jnp.int32),
            jax.ShapeDtypeStruct((K, n), jnp.float32),
        ],
        compiler_params=pltpu.CompilerParams(
            dimension_semantics=("parallel",),
        ),
    )(hs, weight, b)
    return (idx.T, wgt.T)
